# baseline (speedup 1.0000x reference)
import functools

import jax
import jax.numpy as jnp
from jax.experimental import pallas as pl
from jax.experimental.pallas import tpu as pltpu

_EPS = 1e-05
_B = 512
_NIT = 5
_LN_BM = 512
_MM_BM = 128
_VMEM = 58 * 1024 * 1024


def _ln_stats_kernel(nblk, steps, inv_n,
                     x_ref, xn_ref, gram_ref, meanf_ref, cs_ref):
    s = pl.program_id(0)
    xv = x_ref[...]
    d = xv.shape[-1]
    mean = jnp.mean(xv, axis=-1, keepdims=True)
    xc = xv - mean
    var = jnp.sum(xc * xc, axis=-1, keepdims=True) * (1.0 / (d - 1))
    xn = xc / (jnp.sqrt(var) + _EPS)
    xn_ref[...] = xn.astype(jnp.bfloat16)

    @pl.when(s == 0)
    def _():
        gram_ref[...] = jnp.zeros_like(gram_ref)
        cs_ref[...] = jnp.zeros_like(cs_ref)

    g = gram_ref[...]
    for j in range(nblk):
        xj = xn[:, j * _B:(j + 1) * _B]
        g = g + jax.lax.dot_general(
            xj, xj, (((0,), (0,)), ((), ())),
            preferred_element_type=jnp.float32)
    gram_ref[...] = g
    cs_ref[...] += jnp.sum(xn, axis=0, keepdims=True)

    @pl.when(s == steps - 1)
    def _():
        total = cs_ref[...]
        sv = total[:, 0:_B]
        for j in range(1, nblk):
            sv = sv + total[:, j * _B:(j + 1) * _B]
        mrow = (sv * inv_n).astype(jnp.bfloat16)
        meanf_ref[...] = jnp.concatenate([mrow] * nblk, axis=-1)
        cs_ref[0:1, 0:_B] = sv


def _ns_wx_kernel(nblk, n_rows, gram_ref, cs_ref, w_ref, wt_ref, c_scr):
    o = pl.program_id(0)

    @pl.when(o == 0)
    def _():
        g = gram_ref[...]
        s = cs_ref[0:1, 0:_B]
        dim = g.shape[0]
        inv_n = 1.0 / n_rows
        outer = jax.lax.dot_general(
            s, s, (((0,), (0,)), ((), ())),
            preferred_element_type=jnp.float32)
        row = jax.lax.broadcasted_iota(jnp.int32, (dim, dim), 0)
        col = jax.lax.broadcasted_iota(jnp.int32, (dim, dim), 1)
        eye = jnp.where(row == col, jnp.float32(1.0), jnp.float32(0.0))
        cov = g * inv_n - outer * (inv_n * inv_n) + _EPS * eye
        norm_a = jnp.sqrt(jnp.sum(cov * cov))
        y = cov * (1.0 / norm_a)
        z = eye
        for _ in range(_NIT):
            t = 1.5 * eye - 0.5 * jnp.dot(z, y,
                                          preferred_element_type=jnp.float32)
            y = jnp.dot(y, t, preferred_element_type=jnp.float32)
            z = jnp.dot(t, z, preferred_element_type=jnp.float32)
        c_scr[...] = z * jax.lax.rsqrt(norm_a)

    @pl.when(o > 0)
    def _():
        c = c_scr[...]
        for j in range(nblk):
            wj = w_ref[:, j * _B:(j + 1) * _B]
            r = jax.lax.dot_general(
                c, wj, (((0,), (1,)), ((), ())),
                preferred_element_type=jnp.float32)
            wt_ref[j * _B:(j + 1) * _B, :] = r.astype(jnp.bfloat16)


def _final_mm_kernel(x_ref, w_ref, m_ref, b_ref, o_ref):
    xc = x_ref[...] - m_ref[...]
    o_ref[...] = jnp.dot(
        xc, w_ref[...], preferred_element_type=jnp.float32) + b_ref[...]


@jax.jit
def kernel(x, weight, bias):
    n_rows_x, d = x.shape
    d_out = weight.shape[0]
    nblk = d // _B
    n = (n_rows_x * d) // _B
    steps = n_rows_x // _LN_BM
    bo = d_out // nblk

    xn, gram, mean_full, cs = pl.pallas_call(
        functools.partial(_ln_stats_kernel, nblk, steps, 1.0 / n),
        grid=(steps,),
        in_specs=[pl.BlockSpec((_LN_BM, d), lambda s: (s, 0))],
        out_specs=[
            pl.BlockSpec((_LN_BM, d), lambda s: (s, 0)),
            pl.BlockSpec((_B, _B), lambda s: (0, 0)),
            pl.BlockSpec((1, d), lambda s: (0, 0)),
            pl.BlockSpec((1, d), lambda s: (0, 0)),
        ],
        out_shape=[
            jax.ShapeDtypeStruct((n_rows_x, d), jnp.bfloat16),
            jax.ShapeDtypeStruct((_B, _B), jnp.float32),
            jax.ShapeDtypeStruct((1, d), jnp.bfloat16),
            jax.ShapeDtypeStruct((1, d), jnp.float32),
        ],
        compiler_params=pltpu.CompilerParams(
            dimension_semantics=("arbitrary",),
            vmem_limit_bytes=_VMEM,
        ),
        name="ln_stats",
    )(x)

    wt = pl.pallas_call(
        functools.partial(_ns_wx_kernel, nblk, float(n)),
        grid=(nblk + 1,),
        in_specs=[
            pl.BlockSpec((_B, _B), lambda o: (0, 0)),
            pl.BlockSpec((1, d), lambda o: (0, 0)),
            pl.BlockSpec((bo, d), lambda o: (jnp.maximum(o - 1, 0), 0)),
        ],
        out_specs=pl.BlockSpec((d, bo), lambda o: (0, jnp.maximum(o - 1, 0))),
        out_shape=jax.ShapeDtypeStruct((d, d_out), jnp.bfloat16),
        scratch_shapes=[pltpu.VMEM((_B, _B), jnp.float32)],
        compiler_params=pltpu.CompilerParams(
            dimension_semantics=("arbitrary",),
            vmem_limit_bytes=_VMEM,
        ),
        name="ns_wx",
    )(gram, cs, weight)

    bias_row = bias.reshape(1, d_out)

    out = pl.pallas_call(
        _final_mm_kernel,
        grid=(n_rows_x // _MM_BM,),
        in_specs=[
            pl.BlockSpec((_MM_BM, d), lambda i: (i, 0)),
            pl.BlockSpec((d, d_out), lambda i: (0, 0)),
            pl.BlockSpec((1, d), lambda i: (0, 0)),
            pl.BlockSpec((1, d_out), lambda i: (0, 0)),
        ],
        out_specs=pl.BlockSpec((_MM_BM, d_out), lambda i: (i, 0)),
        out_shape=jax.ShapeDtypeStruct((n_rows_x, d_out), jnp.float32),
        compiler_params=pltpu.CompilerParams(
            dimension_semantics=("arbitrary",),
            vmem_limit_bytes=_VMEM,
        ),
        name="final_mm",
    )(xn, wt, mean_full, bias_row)
    return out


# speedup vs baseline: 1.0408x; 1.0408x over previous
import functools

import jax
import jax.numpy as jnp
from jax.experimental import pallas as pl
from jax.experimental.pallas import tpu as pltpu

_EPS = 1e-05
_B = 512
_NIT = 5
_LN_BM = 512
_MM_BM = 256
_VMEM = 58 * 1024 * 1024


def _ln_stats_kernel(nblk, steps, inv_n,
                     x_ref, xn_ref, gram_ref, meanf_ref, cs_ref):
    s = pl.program_id(0)
    d = x_ref.shape[1]
    ch = 256

    @pl.when(s == 0)
    def _():
        gram_ref[...] = jnp.zeros_like(gram_ref)
        cs_ref[...] = jnp.zeros_like(cs_ref)

    g = gram_ref[...]
    csum = jnp.zeros((1, d), jnp.float32)
    for ci in range(_LN_BM // ch):
        xv = x_ref[ci * ch:(ci + 1) * ch, :]
        mean = jnp.mean(xv, axis=-1, keepdims=True)
        xc = xv - mean
        var = jnp.sum(xc * xc, axis=-1, keepdims=True) * (1.0 / (d - 1))
        xn = xc / (jnp.sqrt(var) + _EPS)
        xnb = xn.astype(jnp.bfloat16)
        xn_ref[ci * ch:(ci + 1) * ch, :] = xnb
        for j in range(nblk):
            xj = xnb[:, j * _B:(j + 1) * _B]
            g = g + jax.lax.dot_general(
                xj, xj, (((0,), (0,)), ((), ())),
                preferred_element_type=jnp.float32)
        csum = csum + jnp.sum(xn, axis=0, keepdims=True)
    gram_ref[...] = g
    cs_ref[0:1, :] += csum

    @pl.when(s == steps - 1)
    def _():
        total = cs_ref[0:1, :]
        sv = total[:, 0:_B]
        for j in range(1, nblk):
            sv = sv + total[:, j * _B:(j + 1) * _B]
        mrow = (sv * inv_n).astype(jnp.bfloat16)
        meanf_ref[...] = jnp.concatenate([mrow] * nblk, axis=-1)
        cs_ref[0:1, 0:_B] = sv


def _ns_wx_kernel(nblk, n_rows, gram_ref, cs_ref, w_ref, wt_ref, c_scr):
    o = pl.program_id(0)

    @pl.when(o == 0)
    def _():
        g = gram_ref[...]
        s = cs_ref[0:1, 0:_B]
        dim = g.shape[0]
        inv_n = 1.0 / n_rows
        outer = jax.lax.dot_general(
            s, s, (((0,), (0,)), ((), ())),
            preferred_element_type=jnp.float32)
        row = jax.lax.broadcasted_iota(jnp.int32, (dim, dim), 0)
        col = jax.lax.broadcasted_iota(jnp.int32, (dim, dim), 1)
        eye = jnp.where(row == col, jnp.float32(1.0), jnp.float32(0.0))
        cov = g * inv_n - outer * (inv_n * inv_n) + _EPS * eye
        norm_a = jnp.sqrt(jnp.sum(cov * cov))
        y = cov * (1.0 / norm_a)
        z = eye
        for _ in range(_NIT):
            t = 1.5 * eye - 0.5 * jnp.dot(z, y,
                                          preferred_element_type=jnp.float32)
            y = jnp.dot(y, t, preferred_element_type=jnp.float32)
            z = jnp.dot(t, z, preferred_element_type=jnp.float32)
        c_scr[...] = z * jax.lax.rsqrt(norm_a)

    @pl.when(o > 0)
    def _():
        c = c_scr[...]
        for j in range(nblk):
            wj = w_ref[:, j * _B:(j + 1) * _B]
            r = jax.lax.dot_general(
                c, wj, (((0,), (1,)), ((), ())),
                preferred_element_type=jnp.float32)
            wt_ref[j * _B:(j + 1) * _B, :] = r.astype(jnp.bfloat16)


def _final_mm_kernel(x_ref, w_ref, m_ref, b_ref, o_ref):
    xc = x_ref[...] - m_ref[...]
    o_ref[...] = jnp.dot(
        xc, w_ref[...], preferred_element_type=jnp.float32) + b_ref[...]


@jax.jit
def kernel(x, weight, bias):
    n_rows_x, d = x.shape
    d_out = weight.shape[0]
    nblk = d // _B
    n = (n_rows_x * d) // _B
    steps = n_rows_x // _LN_BM
    bo = d_out // nblk

    xn, gram, mean_full, cs = pl.pallas_call(
        functools.partial(_ln_stats_kernel, nblk, steps, 1.0 / n),
        grid=(steps,),
        in_specs=[pl.BlockSpec((_LN_BM, d), lambda s: (s, 0))],
        out_specs=[
            pl.BlockSpec((_LN_BM, d), lambda s: (s, 0)),
            pl.BlockSpec((_B, _B), lambda s: (0, 0)),
            pl.BlockSpec((1, d), lambda s: (0, 0)),
            pl.BlockSpec((8, d), lambda s: (0, 0)),
        ],
        out_shape=[
            jax.ShapeDtypeStruct((n_rows_x, d), jnp.bfloat16),
            jax.ShapeDtypeStruct((_B, _B), jnp.float32),
            jax.ShapeDtypeStruct((1, d), jnp.bfloat16),
            jax.ShapeDtypeStruct((8, d), jnp.float32),
        ],
        compiler_params=pltpu.CompilerParams(
            dimension_semantics=("arbitrary",),
            vmem_limit_bytes=_VMEM,
        ),
        name="ln_stats",
    )(x)

    wt = pl.pallas_call(
        functools.partial(_ns_wx_kernel, nblk, float(n)),
        grid=(nblk + 1,),
        in_specs=[
            pl.BlockSpec((_B, _B), lambda o: (0, 0)),
            pl.BlockSpec((8, d), lambda o: (0, 0)),
            pl.BlockSpec((bo, d), lambda o: (jnp.maximum(o - 1, 0), 0)),
        ],
        out_specs=pl.BlockSpec((d, bo), lambda o: (0, jnp.maximum(o - 1, 0))),
        out_shape=jax.ShapeDtypeStruct((d, d_out), jnp.bfloat16),
        scratch_shapes=[pltpu.VMEM((_B, _B), jnp.float32)],
        compiler_params=pltpu.CompilerParams(
            dimension_semantics=("arbitrary",),
            vmem_limit_bytes=_VMEM,
        ),
        name="ns_wx",
    )(gram, cs, weight)

    bias_row = bias.reshape(1, d_out)

    out = pl.pallas_call(
        _final_mm_kernel,
        grid=(n_rows_x // _MM_BM,),
        in_specs=[
            pl.BlockSpec((_MM_BM, d), lambda i: (i, 0)),
            pl.BlockSpec((d, d_out), lambda i: (0, 0)),
            pl.BlockSpec((1, d), lambda i: (0, 0)),
            pl.BlockSpec((1, d_out), lambda i: (0, 0)),
        ],
        out_specs=pl.BlockSpec((_MM_BM, d_out), lambda i: (i, 0)),
        out_shape=jax.ShapeDtypeStruct((n_rows_x, d_out), jnp.float32),
        compiler_params=pltpu.CompilerParams(
            dimension_semantics=("arbitrary",),
            vmem_limit_bytes=_VMEM,
        ),
        name="final_mm",
    )(xn, wt, mean_full, bias_row)
    return out
